# baseline (speedup 1.0000x reference)
import functools

import jax
import jax.numpy as jnp
from jax import lax
from jax.experimental import pallas as pl
from jax.experimental.pallas import tpu as pltpu
from jax.experimental.pallas import tpu_sc as plsc

N_HB = 26
VOCAB = 100000
EMBED = 32
BATCH = 4096

NC, NS = 2, 16
NW = NC * NS
ROWS = BATCH * N_HB
CHUNK = ROWS // NW

_mesh = plsc.VectorSubcoreMesh(core_axis_name="c", subcore_axis_name="s")


@functools.partial(
    pl.kernel,
    mesh=_mesh,
    out_type=jax.ShapeDtypeStruct((ROWS, EMBED), jnp.float32),
    scratch_types=[
        pltpu.VMEM((CHUNK,), jnp.int32),
        pltpu.VMEM((CHUNK, EMBED), jnp.float32),
        pltpu.SemaphoreType.DMA,
    ],
    compiler_params=pltpu.CompilerParams(use_tc_tiling_on_sc=False),
)
def _gather(ids_hbm, table_hbm, out_hbm, idx_v, rows_v, sem):
    wid = lax.axis_index("s") * NC + lax.axis_index("c")
    base = wid * CHUNK
    pltpu.sync_copy(ids_hbm.at[pl.ds(base, CHUNK)], idx_v)
    pltpu.async_copy(table_hbm.at[idx_v], rows_v, sem).wait()
    pltpu.sync_copy(rows_v, out_hbm.at[pl.ds(base, CHUNK)])


def kernel(hash_ids, tables):
    ids = hash_ids.astype(jnp.int32)
    flat_ids = (ids + (jnp.arange(N_HB, dtype=jnp.int32) * VOCAB)[None, :]).reshape(-1)
    flat_table = tables.reshape(N_HB * VOCAB, EMBED)
    out = _gather(flat_ids, flat_table)
    return out.reshape(BATCH, N_HB, EMBED)


# speedup vs baseline: 1.0003x; 1.0003x over previous
import functools

import jax
import jax.numpy as jnp
from jax import lax
from jax.experimental import pallas as pl
from jax.experimental.pallas import tpu as pltpu
from jax.experimental.pallas import tpu_sc as plsc

N_HB = 26
VOCAB = 100000
EMBED = 32
BATCH = 4096

NC, NS = 2, 16
NW = NC * NS
ROWS = BATCH * N_HB
CHUNK = ROWS // NW
K = 8
CS = CHUNK // K

_mesh = plsc.VectorSubcoreMesh(core_axis_name="c", subcore_axis_name="s")


@functools.partial(
    pl.kernel,
    mesh=_mesh,
    out_type=jax.ShapeDtypeStruct((ROWS, EMBED), jnp.float32),
    scratch_types=[
        pltpu.VMEM((CHUNK,), jnp.int32),
        pltpu.VMEM((CHUNK, EMBED), jnp.float32),
        pltpu.SemaphoreType.DMA,
    ],
    compiler_params=pltpu.CompilerParams(use_tc_tiling_on_sc=False),
)
def _gather(ids_hbm, table_hbm, out_hbm, idx_v, rows_v, sem):
    wid = lax.axis_index("s") * NC + lax.axis_index("c")
    base = wid * CHUNK
    pltpu.sync_copy(ids_hbm.at[pl.ds(base, CHUNK)], idx_v)
    copies = [
        pltpu.async_copy(
            table_hbm.at[idx_v.at[pl.ds(j * CS, CS)]],
            rows_v.at[pl.ds(j * CS, CS)],
            sem,
        )
        for j in range(K)
    ]
    for c in copies:
        c.wait()
    pltpu.sync_copy(rows_v, out_hbm.at[pl.ds(base, CHUNK)])


def kernel(hash_ids, tables):
    ids = hash_ids.astype(jnp.int32)
    flat_ids = (ids + (jnp.arange(N_HB, dtype=jnp.int32) * VOCAB)[None, :]).reshape(-1)
    flat_table = tables.reshape(N_HB * VOCAB, EMBED)
    out = _gather(flat_ids, flat_table)
    return out.reshape(BATCH, N_HB, EMBED)


# speedup vs baseline: 5.9096x; 5.9076x over previous
import functools

import jax
import jax.numpy as jnp
from jax import lax
from jax.experimental import pallas as pl
from jax.experimental.pallas import tpu as pltpu
from jax.experimental.pallas import tpu_sc as plsc

N_HB = 26
VOCAB = 100000
EMBED = 32
BATCH = 4096

NC, NS = 2, 16

_mesh = plsc.VectorSubcoreMesh(core_axis_name="c", subcore_axis_name="s")


@functools.partial(
    pl.kernel,
    mesh=_mesh,
    out_type=jax.ShapeDtypeStruct((N_HB, EMBED, BATCH), jnp.float32),
    scratch_types=[
        pltpu.VMEM((BATCH,), jnp.int32),
        pltpu.VMEM((VOCAB,), jnp.float32),
        pltpu.VMEM((BATCH,), jnp.float32),
    ],
    compiler_params=pltpu.CompilerParams(needs_layout_passes=False),
)
def _gather(ids_hbm, table_hbm, out_hbm, idx_v, row_v, ob_v):
    d = lax.axis_index("s") * NC + lax.axis_index("c")

    def body(h, _):
        pltpu.sync_copy(ids_hbm.at[h], idx_v)
        pltpu.sync_copy(table_hbm.at[h, d], row_v)

        def blk(i, _):
            idx = idx_v[pl.ds(i * 16, 16)]
            ob_v[pl.ds(i * 16, 16)] = plsc.load_gather(row_v, [idx])
            return ()

        lax.fori_loop(0, BATCH // 16, blk, ())
        pltpu.sync_copy(ob_v, out_hbm.at[h, d])
        return ()

    lax.fori_loop(0, N_HB, body, ())


def kernel(hash_ids, tables):
    ids_t = hash_ids.astype(jnp.int32).T
    tables_t = jnp.transpose(tables, (0, 2, 1))
    out_t = _gather(ids_t, tables_t)
    return jnp.transpose(out_t, (2, 0, 1))
